# Initial kernel scaffold; baseline (speedup 1.0000x reference)
import jax
import jax.numpy as jnp
from jax.experimental import pallas as pl


def kernel(x, gate_W, gate_b, W1, b1, W2, b2):
    raise NotImplementedError("write your pallas kernel here")


# speedup vs baseline: 1.3213x; 1.3213x over previous
import functools

import jax
import jax.numpy as jnp
from jax import lax
from jax.experimental import pallas as pl
from jax.experimental.pallas import tpu as pltpu
from jax.experimental.pallas import tpu_sc as plsc

N = 2048
D = 768
E = 8
K = 2
F = 4 * D
BLK = 128
G = (N * K) // BLK + E
NB = G * BLK

NC, NS = 2, 16
NW = NC * NS
TPW = N // NW
LANES = 16
VPR = D // LANES


def _gate_route_body(x_ref, gw_ref, gb_ref, tw_ref, pos_ref, be_ref):
    xv = x_ref[...]
    logits = jnp.dot(xv, gw_ref[...], preferred_element_type=jnp.float32)
    logits = logits + gb_ref[...]
    m = jnp.max(logits, axis=1, keepdims=True)
    ex = jnp.exp(logits - m)
    p = ex / jnp.sum(ex, axis=1, keepdims=True)

    col = lax.broadcasted_iota(jnp.int32, (N, E), 1)
    m0 = jnp.max(p, axis=1, keepdims=True)
    i0 = jnp.min(jnp.where(p == m0, col, E), axis=1, keepdims=True)
    oh0 = (col == i0)
    p1 = jnp.where(oh0, -jnp.inf, p)
    m1 = jnp.max(p1, axis=1, keepdims=True)
    i1 = jnp.min(jnp.where(p1 == m1, col, E), axis=1, keepdims=True)
    oh1 = (col == i1)
    z128 = jnp.zeros((N, 128), jnp.float32)
    tw_ref[...] = jnp.concatenate([m0 + z128, m1 + z128], axis=1)

    ohf = (oh0 | oh1).astype(jnp.float32)
    c = ohf
    sh = 1
    while sh < N:
        c = c + jnp.concatenate([jnp.zeros((sh, E), jnp.float32), c[: N - sh]], axis=0)
        sh *= 2
    excl = c - ohf
    counts = jnp.sum(ohf, axis=0, keepdims=True)
    ci = counts.astype(jnp.int32)
    pc = ((ci + BLK - 1) // BLK) * BLK
    inc = pc
    sh = 1
    while sh < E:
        inc = inc + jnp.concatenate(
            [jnp.zeros((1, sh), jnp.int32), inc[:, : E - sh]], axis=1)
        sh *= 2
    off = (inc - pc).astype(jnp.float32)

    oh0f = oh0.astype(jnp.float32)
    oh1f = oh1.astype(jnp.float32)
    r0 = jnp.sum(excl * oh0f, axis=1, keepdims=True)
    r1 = jnp.sum(excl * oh1f, axis=1, keepdims=True)
    s0 = jnp.sum(off * oh0f, axis=1, keepdims=True)
    s1 = jnp.sum(off * oh1f, axis=1, keepdims=True)
    pos_ref[...] = jnp.concatenate([s0 + r0, s1 + r1], axis=1).astype(jnp.int32)

    bi = lax.broadcasted_iota(jnp.int32, (1, G), 1) * BLK
    acc = jnp.zeros((1, G), jnp.int32)
    offi = (inc - pc)
    for e in range(E):
        acc = acc + jnp.where(bi >= offi[:, e:e + 1], 1, 0)
    be_ref[...] = acc - 1


def _ffn_body(be_ref, x_ref, w1_ref, b1_ref, w2_ref, b2_ref, ws_ref, y_ref):
    del be_ref
    h = jnp.dot(x_ref[...], w1_ref[0], preferred_element_type=jnp.float32)
    h = jnp.maximum(h + b1_ref[0], 0.0)
    y = jnp.dot(h, w2_ref[0], preferred_element_type=jnp.float32) + b2_ref[0]
    y_ref[...] = y * ws_ref[:, :1]


def _grouped_ffn(be, xs, W1, b1, W2, b2, ws):
    grid_spec = pltpu.PrefetchScalarGridSpec(
        num_scalar_prefetch=1,
        grid=(G,),
        in_specs=[
            pl.BlockSpec((BLK, D), lambda g, be: (g, 0)),
            pl.BlockSpec((1, D, F), lambda g, be: (be[g], 0, 0)),
            pl.BlockSpec((1, 1, F), lambda g, be: (be[g], 0, 0)),
            pl.BlockSpec((1, F, D), lambda g, be: (be[g], 0, 0)),
            pl.BlockSpec((1, 1, D), lambda g, be: (be[g], 0, 0)),
            pl.BlockSpec((BLK, 128), lambda g, be: (g, 0)),
        ],
        out_specs=pl.BlockSpec((BLK, D), lambda g, be: (g, 0)),
    )
    return pl.pallas_call(
        _ffn_body,
        grid_spec=grid_spec,
        out_shape=jax.ShapeDtypeStruct((NB, D), jnp.float32),
        compiler_params=pltpu.CompilerParams(
            dimension_semantics=("arbitrary",)),
    )(be, xs, W1, b1.reshape(E, 1, F), W2, b2.reshape(E, 1, D), ws)


@functools.cache
def _sc_kernels():
    mesh = plsc.VectorSubcoreMesh(core_axis_name="c", subcore_axis_name="s")

    @functools.partial(
        pl.kernel,
        mesh=mesh,
        out_type=(
            jax.ShapeDtypeStruct((NB, D), jnp.float32),
            jax.ShapeDtypeStruct((NB, 128), jnp.float32),
        ),
        scratch_types=[
            pltpu.VMEM((TPW, D), jnp.float32),
            pltpu.VMEM((TPW, 128), jnp.float32),
            pltpu.VMEM((TPW,), jnp.int32),
        ],
    )
    def sc_scatter(x_hbm, pos_hbm, wt_hbm, xs_hbm, ws_hbm, xr_v, wv_v, p_v):
        w = lax.axis_index("s") * NC + lax.axis_index("c")
        pltpu.sync_copy(x_hbm.at[pl.ds(w * TPW, TPW)], xr_v)
        for k in range(K):
            pltpu.sync_copy(pos_hbm.at[k * NW + w], p_v)
            pltpu.sync_copy(wt_hbm.at[k * NW + w], wv_v)
            pltpu.sync_copy(xr_v, xs_hbm.at[p_v])
            pltpu.sync_copy(wv_v, ws_hbm.at[p_v])

    @functools.partial(
        pl.kernel,
        mesh=mesh,
        out_type=jax.ShapeDtypeStruct((N, D), jnp.float32),
        scratch_types=[
            pltpu.VMEM((TPW, D), jnp.float32),
            pltpu.VMEM((TPW, D), jnp.float32),
            pltpu.VMEM((TPW,), jnp.int32),
        ],
    )
    def sc_combine(ys_hbm, pos_hbm, out_hbm, r_v, o_v, p_v):
        w = lax.axis_index("s") * NC + lax.axis_index("c")

        pltpu.sync_copy(pos_hbm.at[w], p_v)
        pltpu.sync_copy(ys_hbm.at[p_v], o_v)
        pltpu.sync_copy(pos_hbm.at[NW + w], p_v)
        pltpu.sync_copy(ys_hbm.at[p_v], r_v)

        def addrow(i, _):
            for v in range(VPR):
                sl = pl.ds(v * LANES, LANES)
                o_v[i, sl] = o_v[i, sl] + r_v[i, sl]
            return 0

        lax.fori_loop(0, TPW, addrow, 0)
        pltpu.sync_copy(o_v, out_hbm.at[pl.ds(w * TPW, TPW)])

    return sc_scatter, sc_combine


def kernel(x, gate_W, gate_b, W1, b1, W2, b2):
    tw, pos, be = pl.pallas_call(
        _gate_route_body,
        out_shape=(
            jax.ShapeDtypeStruct((N, K * 128), jnp.float32),
            jax.ShapeDtypeStruct((N, K), jnp.int32),
            jax.ShapeDtypeStruct((1, G), jnp.int32),
        ),
    )(x, gate_W, gate_b.reshape(1, E))
    pos_scat = pos.T.reshape(K * NW, TPW)
    wt_scat = (tw.reshape(N, K, 128).transpose(1, 0, 2)
               .reshape(K * NW, TPW, 128))
    be_flat = be.reshape(G)

    sc_scatter, sc_combine = _sc_kernels()
    xs, ws = sc_scatter(x, pos_scat, wt_scat)
    ys = _grouped_ffn(be_flat, xs, W1, b1, W2, b2, ws)
    return sc_combine(ys, pos_scat)
